# baseline (speedup 1.0000x reference)
import jax
import jax.numpy as jnp
from jax import lax
from jax.experimental import pallas as pl
from jax.experimental.pallas import tpu as pltpu
from jax.experimental.pallas import tpu_sc as plsc

RESOLUTION = 128
FEATURE_DIM = 32
NFPD = 16
RADIUS = 1.0
N_INT = 128
STEP = 0.01
WIDTH = 64
BATCH = 4096
N_PTS = BATCH * N_INT

NC = 2
NS = 16
L = 16
NW = NC * NS
PW = N_PTS // NW
CH = 512
NCHUNK = PW // CH
GRP = CH // L

RAY_BLOCK = 128
N_BLOCKS = BATCH // RAY_BLOCK
PB = RAY_BLOCK * N_INT


def _tree8(vs):
    return (vs[0] + vs[1]) + (vs[2] + vs[3]) + ((vs[4] + vs[5]) + (vs[6] + vs[7]))


RP_Y = 16
RP_UNITS = RESOLUTION * (RESOLUTION // RP_Y)
RP_PER_W = RP_UNITS // NW
RP_ROWS = RP_Y * RESOLUTION


def _repack_body(g1_h, t1_h, slabA, slabB, outA, outB, ss0, ss1, so0, so1):
    SLAB = (slabA, slabB)
    OUTB = (outA, outB)
    SS = (ss0, ss1)
    SO = (so0, so1)
    wid = lax.axis_index("s") * NC + lax.axis_index("c")
    lanes = lax.iota(jnp.int32, L)
    M = RESOLUTION - 1

    def params(u):
        ug = wid * RP_PER_W + jnp.minimum(u, RP_PER_W - 1)
        z = ug // (RESOLUTION // RP_Y)
        y0 = (ug % (RESOLUTION // RP_Y)) * RP_Y
        return z, y0, jnp.minimum(y0, RESOLUTION - (RP_Y + 1))

    def slab_descs(u, b, make):
        z, y0, yb = params(u)
        z1 = jnp.minimum(z + 1, M)
        f = pltpu.make_async_copy if make else pltpu.async_copy
        return [f(g1_h.at[c, zc, pl.ds(yb, RP_Y + 1)],
                  SLAB[b].at[c, dz], SS[b])
                for c in range(3) for dz, zc in ((0, z), (1, z1))]

    def drain_slabs(u, b):
        for dsc in slab_descs(u, b, True):
            dsc.wait()

    def out_desc(u, b, make):
        z, y0, _ = params(u)
        f = pltpu.make_async_copy if make else pltpu.async_copy
        return f(OUTB[b], t1_h.at[pl.ds(z * (RESOLUTION * RESOLUTION)
                                        + y0 * RESOLUTION, RP_ROWS)], SO[b])

    def compute(u, b):
        slab, outb = SLAB[b], OUTB[b]
        z, y0, yb = params(u)

        @plsc.parallel_loop(0, RP_Y, unroll=2)
        def row_body(j):
            yj = y0 + j
            i0 = jnp.minimum(yj, M) - yb
            i1 = jnp.minimum(yj + 1, M) - yb
            for xg in range(RESOLUTION // L):
                x_lin = pl.ds(xg * L, L)
                xi1 = jnp.minimum(xg * L + lanes + 1, M)
                orow = j * RESOLUTION + xg * L + lanes
                for dy, iy in ((0, i0), (1, i1)):
                    iyv = jnp.full((L,), iy, jnp.int32)
                    for dx in (0, 1):
                        q = dy * 2 + dx
                        for c in range(3):
                            if dx == 0:
                                a = slab[c, 0, iy, x_lin]
                                bb = slab[c, 1, iy, x_lin]
                            else:
                                cv = jnp.full((L,), c, jnp.int32)
                                z0v = jnp.full((L,), 0, jnp.int32)
                                z1v = jnp.full((L,), 1, jnp.int32)
                                a = plsc.load_gather(slab, [cv, z0v, iyv, xi1])
                                bb = plsc.load_gather(slab, [cv, z1v, iyv, xi1])
                            pk = plsc.bitcast(
                                plsc.pack(a, bb,
                                          format=plsc.PackFormat.INTERLEAVED),
                                jnp.int32)
                            plsc.store_scatter(
                                outb,
                                [orow, jnp.full((L,), q * 3 + c, jnp.int32)],
                                pk)

    slab_descs(0, 0, False)
    slab_descs(1, 1, False)

    def pair_body(p, carry):
        for b in (0, 1):
            u = p * 2 + b
            drain_slabs(u, b)

            @pl.when(p >= 1)
            def _():
                out_desc(u - 2, b, True).wait()

            compute(u, b)
            out_desc(u, b, False)
            slab_descs(u + 2, b, False)
        return carry
    lax.fori_loop(0, RP_PER_W // 2, pair_body, 0)
    drain_slabs(RP_PER_W, 0)
    drain_slabs(RP_PER_W + 1, 1)
    out_desc(RP_PER_W - 2, 0, True).wait()
    out_desc(RP_PER_W - 1, 1, True).wait()


def _build_g1_table(G1):
    mesh = plsc.VectorSubcoreMesh(core_axis_name="c", subcore_axis_name="s")
    fn = pl.kernel(
        _repack_body,
        out_type=jax.ShapeDtypeStruct((RESOLUTION ** 3, 16), jnp.int32),
        mesh=mesh,
        compiler_params=pltpu.CompilerParams(
            needs_layout_passes=False, use_tc_tiling_on_sc=False),
        scratch_types=(
            [pltpu.VMEM((3, 2, RP_Y + 1, RESOLUTION), jnp.float32)] * 2
            + [pltpu.VMEM((RP_ROWS, 16), jnp.int32)] * 2
            + [pltpu.SemaphoreType.DMA] * 4
        ),
    )
    return fn(G1)


def _build_f_table(F):
    f = jnp.transpose(F, (1, 2, 3, 0)).reshape(NFPD ** 3, FEATURE_DIM)
    fb = f.astype(jnp.bfloat16).reshape(NFPD ** 3, FEATURE_DIM // 2, 2)
    return lax.bitcast_convert_type(fb, jnp.int32)


def _sc_body(px_h, py_h, pz_h, t1_h, fp_h, out_h,
             bx0, by0, bz0, bx1, by1, bz1, idx0, idx1, rows0, rows1,
             outb0, outb1, ftab, sp0, sp1, sr0, sr1, so0, so1):
    BX = (bx0, bx1)
    BY = (by0, by1)
    BZ = (bz0, bz1)
    IDX = (idx0, idx1)
    ROWS = (rows0, rows1)
    OUTB = (outb0, outb1)
    SP = (sp0, sp1)
    SR = (sr0, sr1)
    SO = (so0, so1)
    wid = lax.axis_index("s") * NC + lax.axis_index("c")
    base_w = wid * PW
    pltpu.sync_copy(fp_h, ftab)
    lanes = lax.iota(jnp.int32, L)
    G1M = RESOLUTION - 1
    G1S = (RESOLUTION - 1) * 0.5
    FM = NFPD - 1
    FS = (NFPD - 1) * 0.5

    def cbase(u):
        return base_w + jnp.minimum(u, NCHUNK - 1) * CH

    def pts_descs(u, b, make):
        base = cbase(u)
        f = pltpu.make_async_copy if make else pltpu.async_copy
        return [f(px_h.at[pl.ds(base, CH)], BX[b], SP[b]),
                f(py_h.at[pl.ds(base, CH)], BY[b], SP[b]),
                f(pz_h.at[pl.ds(base, CH)], BZ[b], SP[b])]

    def fire_pts(u, b):
        pts_descs(u, b, False)

    def drain_pts(u, b):
        for dsc in pts_descs(u, b, True):
            dsc.wait()

    def idx_compute(b):
        bx, by, bz, idxb = BX[b], BY[b], BZ[b], IDX[b]

        @plsc.parallel_loop(0, GRP, unroll=4)
        def idx_body(g):
            s = pl.ds(pl.multiple_of(g * L, L), L)
            x0 = jnp.clip((bx[s] * G1S + G1S).astype(jnp.int32), 0, G1M)
            y0 = jnp.clip((by[s] * G1S + G1S).astype(jnp.int32), 0, G1M)
            z0 = jnp.clip((bz[s] * G1S + G1S).astype(jnp.int32), 0, G1M)
            idxb[s] = x0 + y0 * RESOLUTION + z0 * (RESOLUTION * RESOLUTION)

    def row_descs(b, make):
        f = pltpu.make_async_copy if make else pltpu.async_copy
        return [f(t1_h.at[IDX[b].at[pl.ds(j * 128, 128)]],
                  ROWS[b].at[pl.ds(j * 128, 128)], SR[b])
                for j in range(CH // 128)]

    def fire_rows(b):
        row_descs(b, False)

    def drain_rows(b):
        for j in range(CH // 128):
            pltpu.make_async_copy(t1_h.at[pl.ds(0, 128)],
                                  ROWS[b].at[pl.ds(j * 128, 128)],
                                  SR[b]).wait()

    def out_desc(u, b, make):
        f = pltpu.make_async_copy if make else pltpu.async_copy
        return f(OUTB[b], out_h.at[:, pl.ds(cbase(u), CH)], SO[b])

    def main_compute(b):
        bx, by, bz, rows, outb = BX[b], BY[b], BZ[b], ROWS[b], OUTB[b]

        @plsc.parallel_loop(0, GRP, unroll=4)
        def grp_body(g):
            off = pl.multiple_of(g * L, L)
            s = pl.ds(off, L)
            rowbase = g * L + lanes
            cx = bx[s] * G1S + G1S
            cy = by[s] * G1S + G1S
            cz = bz[s] * G1S + G1S
            ix = cx.astype(jnp.int32)
            iy = cy.astype(jnp.int32)
            iz = cz.astype(jnp.int32)
            fx = cx - ix.astype(jnp.float32)
            fy = cy - iy.astype(jnp.float32)
            fz = cz - iz.astype(jnp.float32)
            wq = (
                (1.0 - fy) * (1.0 - fx),
                (1.0 - fy) * fx,
                fy * (1.0 - fx),
                fy * fx,
            )
            wz1 = fz
            wz0 = 1.0 - fz
            wA = [wq[q] * wz0 for q in range(4)]
            wB = [wq[q] * wz1 for q in range(4)]
            g1 = []
            for c in range(3):
                terms = []
                for q in range(4):
                    wv = plsc.load_gather(
                        rows, [rowbase, jnp.full((L,), q * 3 + c, jnp.int32)])
                    a, b = plsc.unpack(plsc.bitcast(wv, jnp.bfloat16),
                                       format=plsc.PackFormat.INTERLEAVED)
                    terms.append(wA[q] * a + wB[q] * b)
                g1.append((terms[0] + terms[1]) + (terms[2] + terms[3]))

            cfx = g1[0] * FS + FS
            cfy = g1[1] * FS + FS
            cfz = g1[2] * FS + FS
            jx = cfx.astype(jnp.int32)
            jy = cfy.astype(jnp.int32)
            jz = cfz.astype(jnp.int32)
            ffx = cfx - jx.astype(jnp.float32)
            ffy = cfy - jy.astype(jnp.float32)
            ffz = cfz - jz.astype(jnp.float32)
            xs = (jnp.clip(jx, 0, FM),)
            xs = xs + (jnp.minimum(xs[0] + 1, FM),)
            ys = (jnp.clip(jy, 0, FM),)
            ys = ys + (jnp.minimum(ys[0] + 1, FM),)
            zs = (jnp.clip(jz, 0, FM),)
            zs = zs + (jnp.minimum(zs[0] + 1, FM),)
            wxs = (1.0 - ffx, ffx)
            wys = (1.0 - ffy, ffy)
            wzs = (1.0 - ffz, ffz)
            cells = []
            wfs = []
            for dz in (0, 1):
                for dy in (0, 1):
                    for dx in (0, 1):
                        cells.append(xs[dx] + ys[dy] * NFPD
                                     + zs[dz] * (NFPD * NFPD))
                        wfs.append(wzs[dz] * wys[dy] * wxs[dx])
            for w in range(FEATURE_DIM // 2):
                wcol = jnp.full((L,), w, jnp.int32)
                ta = []
                tb = []
                for k in range(8):
                    wv = plsc.load_gather(ftab, [cells[k], wcol])
                    a, b = plsc.unpack(plsc.bitcast(wv, jnp.bfloat16),
                                       format=plsc.PackFormat.INTERLEAVED)
                    ta.append(wfs[k] * a)
                    tb.append(wfs[k] * b)
                outb[2 * w, s] = _tree8(ta)
                outb[2 * w + 1, s] = _tree8(tb)

    fire_pts(0, 0)
    drain_pts(0, 0)
    idx_compute(0)
    fire_rows(0)
    fire_pts(1, 1)

    def pair_body(p, carry):
        for b in (0, 1):
            u = p * 2 + b
            ob = 1 - b
            drain_pts(u + 1, ob)
            idx_compute(ob)
            fire_rows(ob)
            drain_rows(b)

            @pl.when(p >= 1)
            def _():
                out_desc(u - 2, b, True).wait()

            main_compute(b)
            out_desc(u, b, False)
            fire_pts(u + 2, b)
        return carry
    lax.fori_loop(0, NCHUNK // 2, pair_body, 0)
    drain_rows(0)
    drain_pts(NCHUNK + 1, 1)
    out_desc(NCHUNK - 2, 0, True).wait()
    out_desc(NCHUNK - 1, 1, True).wait()


def _sc_gather(px, py, pz, T1, Fp):
    mesh = plsc.VectorSubcoreMesh(core_axis_name="c", subcore_axis_name="s")
    fn = pl.kernel(
        _sc_body,
        out_type=jax.ShapeDtypeStruct((FEATURE_DIM, N_PTS), jnp.float32),
        mesh=mesh,
        compiler_params=pltpu.CompilerParams(
            needs_layout_passes=False, use_tc_tiling_on_sc=False),
        scratch_types=(
            [pltpu.VMEM((CH,), jnp.float32)] * 6
            + [pltpu.VMEM((CH,), jnp.int32)] * 2
            + [pltpu.VMEM((CH, 16), jnp.int32)] * 2
            + [pltpu.VMEM((FEATURE_DIM, CH), jnp.float32)] * 2
            + [pltpu.VMEM((NFPD ** 3, FEATURE_DIM // 2), jnp.int32)]
            + [pltpu.SemaphoreType.DMA] * 6
        ),
    )
    return fn(px, py, pz, T1, Fp)


def _shift_right(x, sh):
    col = lax.broadcasted_iota(jnp.int32, x.shape, 1)
    return jnp.where(col >= sh, pltpu.roll(x, sh, axis=1), 0.0)


def _cumsum_minor(x):
    for sh in (1, 2, 4, 8, 16, 32, 64):
        x = x + _shift_right(x, sh)
    return x


def _bmm(a, b):
    return lax.dot(a.astype(jnp.bfloat16), b.astype(jnp.bfloat16),
                   preferred_element_type=jnp.float32)


def _mlp_body(fv_ref, d_ref, m_ref, t_ref, Ws1_ref, bs1_ref, Ws2_ref, bs2_ref,
              Wc1a_ref, Wc1b_ref, bc1_ref, Wc2_ref, bc2_ref, out_ref):
    R = RAY_BLOCK
    fv = fv_ref[...].astype(jnp.bfloat16)
    h = jnp.maximum(_bmm(Ws1_ref[...], fv) + bs1_ref[...][:, None], 0.0)
    sigma = jnp.maximum(_bmm(Ws2_ref[...], h) + bs2_ref[...][:, None], 0.0)[0]
    sigma2 = sigma.reshape(R, N_INT)
    m = m_ref[...]
    sigma2 = jnp.where(m, sigma2, 0.0)
    t_ = t_ref[...]
    col = lax.broadcasted_iota(jnp.int32, t_.shape, 1)
    t_next = pltpu.roll(t_, N_INT - 1, axis=1)
    deltas = jnp.where(col < N_INT - 1, t_next - t_, STEP)
    alpha = 1.0 - jnp.exp(-sigma2 * deltas)
    logx = jnp.log(1.0 - alpha + 1e-10)
    trans = jnp.exp(_shift_right(_cumsum_minor(logx), 1))
    abs_light = alpha * trans
    acc = jnp.sum(abs_light, axis=1)
    hc = _bmm(Wc1a_ref[...], fv)
    dpart = Wc1b_ref[...] @ d_ref[...] + bc1_ref[...][:, None]
    hc = hc.reshape(WIDTH, R, N_INT) + dpart[:, :, None]
    hc = jnp.maximum(hc, 0.0).reshape(WIDTH, PB)
    pre = _bmm(Wc2_ref[...], hc) + bc2_ref[...][:, None]
    outs = []
    for cc in range(3):
        rgb_c = jax.nn.sigmoid(pre[cc].reshape(R, N_INT))
        rgb_c = jnp.where(m, rgb_c, 0.0)
        o_c = jnp.sum(abs_light * rgb_c, axis=1) + (1.0 - acc)
        outs.append(o_c[:, None])
    out_ref[...] = jnp.concatenate(outs, axis=1)


def _mlp_composite(FvalsT, dT, mask, t, Ws1, bs1, Ws2, bs2, Wc1, bc1, Wc2,
                   bc2):
    return pl.pallas_call(
        _mlp_body,
        grid=(N_BLOCKS,),
        in_specs=[
            pl.BlockSpec((FEATURE_DIM, PB), lambda i: (0, i)),
            pl.BlockSpec((3, RAY_BLOCK), lambda i: (0, i)),
            pl.BlockSpec((RAY_BLOCK, N_INT), lambda i: (i, 0)),
            pl.BlockSpec((RAY_BLOCK, N_INT), lambda i: (i, 0)),
            pl.BlockSpec((WIDTH, FEATURE_DIM), lambda i: (0, 0)),
            pl.BlockSpec((WIDTH,), lambda i: (0,)),
            pl.BlockSpec((1, WIDTH), lambda i: (0, 0)),
            pl.BlockSpec((1,), lambda i: (0,)),
            pl.BlockSpec((WIDTH, FEATURE_DIM), lambda i: (0, 0)),
            pl.BlockSpec((WIDTH, 3), lambda i: (0, 0)),
            pl.BlockSpec((WIDTH,), lambda i: (0,)),
            pl.BlockSpec((3, WIDTH), lambda i: (0, 0)),
            pl.BlockSpec((3,), lambda i: (0,)),
        ],
        out_specs=pl.BlockSpec((RAY_BLOCK, 3), lambda i: (i, 0)),
        out_shape=jax.ShapeDtypeStruct((BATCH, 3), jnp.float32),
    )(FvalsT, dT, mask, t, Ws1.T, bs1, Ws2.T, bs2, Wc1[:FEATURE_DIM].T,
      Wc1[FEATURE_DIM:].T, bc1, Wc2.T, bc2)


def kernel(rays_o, rays_d, G1, F, Ws1, bs1, Ws2, bs2, Wc1, bc1, Wc2, bc2):
    d = rays_d / jnp.linalg.norm(rays_d, axis=-1, keepdims=True)
    b = jnp.sum(rays_o * d, axis=-1)
    c = jnp.sum(rays_o * rays_o, axis=-1) - RADIUS * RADIUS
    disc = b * b - c
    t_near = jnp.maximum(-b - jnp.sqrt(jnp.maximum(disc, 0.0)), 0.0)
    t = t_near[:, None] + jnp.arange(N_INT, dtype=jnp.float32)[None, :] * STEP
    pts = rays_o[:, None, :] + t[..., None] * d[:, None, :]
    mask = (disc[:, None] > 0.0) & (jnp.linalg.norm(pts, axis=-1) <= RADIUS)
    pts = pts / RADIUS
    px = pts[..., 0].reshape(-1)
    py = pts[..., 1].reshape(-1)
    pz = pts[..., 2].reshape(-1)

    T1 = _build_g1_table(G1)
    Fp = _build_f_table(F)
    FvalsT = _sc_gather(px, py, pz, T1, Fp)

    return _mlp_composite(FvalsT, d.T, mask, t,
                          Ws1, bs1, Ws2, bs2, Wc1, bc1, Wc2, bc2)


# speedup vs baseline: 1.8109x; 1.8109x over previous
import jax
import jax.numpy as jnp
from jax import lax
from jax.experimental import pallas as pl
from jax.experimental.pallas import tpu as pltpu
from jax.experimental.pallas import tpu_sc as plsc

RESOLUTION = 128
FEATURE_DIM = 32
NFPD = 16
RADIUS = 1.0
N_INT = 128
STEP = 0.01
WIDTH = 64
BATCH = 4096
N_PTS = BATCH * N_INT

NC = 2
NS = 16
L = 16
NW = NC * NS
PW = N_PTS // NW
CH = 512
NCHUNK = PW // CH
GRP = CH // L

RAY_BLOCK = 128
N_BLOCKS = BATCH // RAY_BLOCK
PB = RAY_BLOCK * N_INT


def _tree8(vs):
    return (vs[0] + vs[1]) + (vs[2] + vs[3]) + ((vs[4] + vs[5]) + (vs[6] + vs[7]))


RP_Y = 16
RP_UNITS = RESOLUTION * (RESOLUTION // RP_Y)
RP_PER_W = RP_UNITS // NW
RP_ROWS = RP_Y * RESOLUTION


def _repack_body(g1_h, t1_h, slabA, slabB, outA, outB, ss0, ss1, so0, so1):
    SLAB = (slabA, slabB)
    OUTB = (outA, outB)
    SS = (ss0, ss1)
    SO = (so0, so1)
    wid = lax.axis_index("s") * NC + lax.axis_index("c")
    lanes = lax.iota(jnp.int32, L)
    M = RESOLUTION - 1

    def params(u):
        ug = wid * RP_PER_W + jnp.minimum(u, RP_PER_W - 1)
        z = ug // (RESOLUTION // RP_Y)
        y0 = (ug % (RESOLUTION // RP_Y)) * RP_Y
        return z, y0, jnp.minimum(y0, RESOLUTION - (RP_Y + 1))

    def slab_descs(u, b, make):
        z, y0, yb = params(u)
        z1 = jnp.minimum(z + 1, M)
        f = pltpu.make_async_copy if make else pltpu.async_copy
        return [f(g1_h.at[c, zc, pl.ds(yb, RP_Y + 1)],
                  SLAB[b].at[c, dz], SS[b])
                for c in range(3) for dz, zc in ((0, z), (1, z1))]

    def drain_slabs(u, b):
        for dsc in slab_descs(u, b, True):
            dsc.wait()

    def out_desc(u, b, make):
        z, y0, _ = params(u)
        f = pltpu.make_async_copy if make else pltpu.async_copy
        return f(OUTB[b], t1_h.at[pl.ds(z * (RESOLUTION * RESOLUTION)
                                        + y0 * RESOLUTION, RP_ROWS)], SO[b])

    def compute(u, b):
        slab, outb = SLAB[b], OUTB[b]
        z, y0, yb = params(u)

        @plsc.parallel_loop(0, RP_Y, unroll=2)
        def row_body(j):
            yj = y0 + j
            i0 = jnp.minimum(yj, M) - yb
            i1 = jnp.minimum(yj + 1, M) - yb
            for xg in range(RESOLUTION // L):
                x_lin = pl.ds(xg * L, L)
                xi1 = jnp.minimum(xg * L + lanes + 1, M)
                orow = j * RESOLUTION + xg * L + lanes
                for dy, iy in ((0, i0), (1, i1)):
                    iyv = jnp.full((L,), iy, jnp.int32)
                    for dx in (0, 1):
                        q = dy * 2 + dx
                        for c in range(3):
                            if dx == 0:
                                a = slab[c, 0, iy, x_lin]
                                bb = slab[c, 1, iy, x_lin]
                            else:
                                cv = jnp.full((L,), c, jnp.int32)
                                z0v = jnp.full((L,), 0, jnp.int32)
                                z1v = jnp.full((L,), 1, jnp.int32)
                                a = plsc.load_gather(slab, [cv, z0v, iyv, xi1])
                                bb = plsc.load_gather(slab, [cv, z1v, iyv, xi1])
                            pk = plsc.bitcast(
                                plsc.pack(a, bb,
                                          format=plsc.PackFormat.INTERLEAVED),
                                jnp.int32)
                            plsc.store_scatter(
                                outb,
                                [orow, jnp.full((L,), q * 3 + c, jnp.int32)],
                                pk)

    slab_descs(0, 0, False)
    slab_descs(1, 1, False)

    def pair_body(p, carry):
        for b in (0, 1):
            u = p * 2 + b
            drain_slabs(u, b)

            @pl.when(p >= 1)
            def _():
                out_desc(u - 2, b, True).wait()

            compute(u, b)
            out_desc(u, b, False)
            slab_descs(u + 2, b, False)
        return carry
    lax.fori_loop(0, RP_PER_W // 2, pair_body, 0)
    drain_slabs(RP_PER_W, 0)
    drain_slabs(RP_PER_W + 1, 1)
    out_desc(RP_PER_W - 2, 0, True).wait()
    out_desc(RP_PER_W - 1, 1, True).wait()


def _build_g1_table(G1):
    mesh = plsc.VectorSubcoreMesh(core_axis_name="c", subcore_axis_name="s")
    fn = pl.kernel(
        _repack_body,
        out_type=jax.ShapeDtypeStruct((RESOLUTION ** 3, 16), jnp.int32),
        mesh=mesh,
        compiler_params=pltpu.CompilerParams(
            needs_layout_passes=False, use_tc_tiling_on_sc=False),
        scratch_types=(
            [pltpu.VMEM((3, 2, RP_Y + 1, RESOLUTION), jnp.float32)] * 2
            + [pltpu.VMEM((RP_ROWS, 16), jnp.int32)] * 2
            + [pltpu.SemaphoreType.DMA] * 4
        ),
    )
    return fn(G1)


def _build_f_table(F):
    f = jnp.transpose(F, (1, 2, 3, 0)).reshape(NFPD ** 3, FEATURE_DIM)
    fb = f.astype(jnp.bfloat16).reshape(NFPD ** 3, FEATURE_DIM // 2, 2)
    return lax.bitcast_convert_type(fb, jnp.int32)


def _sc_body(px_h, py_h, pz_h, t1_h, fp_h, out_h,
             bx0, by0, bz0, bx1, by1, bz1, idx0, idx1, rows0, rows1,
             outb0, outb1, ftab, sp0, sp1, sr0, sr1, so0, so1):
    BX = (bx0, bx1)
    BY = (by0, by1)
    BZ = (bz0, bz1)
    IDX = (idx0, idx1)
    ROWS = (rows0, rows1)
    OUTB = (outb0, outb1)
    SP = (sp0, sp1)
    SR = (sr0, sr1)
    SO = (so0, so1)
    wid = lax.axis_index("s") * NC + lax.axis_index("c")
    base_w = wid * PW
    pltpu.sync_copy(fp_h, ftab)
    lanes = lax.iota(jnp.int32, L)
    G1M = RESOLUTION - 1
    G1S = (RESOLUTION - 1) * 0.5
    FM = NFPD - 1
    FS = (NFPD - 1) * 0.5

    def cbase(u):
        return base_w + jnp.minimum(u, NCHUNK - 1) * CH

    def pts_descs(u, b, make):
        base = cbase(u)
        f = pltpu.make_async_copy if make else pltpu.async_copy
        return [f(px_h.at[pl.ds(base, CH)], BX[b], SP[b]),
                f(py_h.at[pl.ds(base, CH)], BY[b], SP[b]),
                f(pz_h.at[pl.ds(base, CH)], BZ[b], SP[b])]

    def fire_pts(u, b):
        pts_descs(u, b, False)

    def drain_pts(u, b):
        for dsc in pts_descs(u, b, True):
            dsc.wait()

    def idx_compute(b):
        bx, by, bz, idxb = BX[b], BY[b], BZ[b], IDX[b]

        @plsc.parallel_loop(0, GRP, unroll=4)
        def idx_body(g):
            s = pl.ds(pl.multiple_of(g * L, L), L)
            x0 = jnp.clip((bx[s] * G1S + G1S).astype(jnp.int32), 0, G1M)
            y0 = jnp.clip((by[s] * G1S + G1S).astype(jnp.int32), 0, G1M)
            z0 = jnp.clip((bz[s] * G1S + G1S).astype(jnp.int32), 0, G1M)
            idxb[s] = x0 + y0 * RESOLUTION + z0 * (RESOLUTION * RESOLUTION)

    def row_descs(b, make):
        f = pltpu.make_async_copy if make else pltpu.async_copy
        return [f(t1_h.at[IDX[b].at[pl.ds(j * 128, 128)]],
                  ROWS[b].at[pl.ds(j * 128, 128)], SR[b])
                for j in range(CH // 128)]

    def fire_rows(b):
        row_descs(b, False)

    def drain_rows(b):
        for j in range(CH // 128):
            pltpu.make_async_copy(t1_h.at[pl.ds(0, 128)],
                                  ROWS[b].at[pl.ds(j * 128, 128)],
                                  SR[b]).wait()

    def out_desc(u, b, make):
        f = pltpu.make_async_copy if make else pltpu.async_copy
        return f(OUTB[b], out_h.at[:, pl.ds(cbase(u), CH)], SO[b])

    def main_compute(b):
        bx, by, bz, rows, outb = BX[b], BY[b], BZ[b], ROWS[b], OUTB[b]

        @plsc.parallel_loop(0, GRP, unroll=2)
        def grp_body(g):
            off = pl.multiple_of(g * L, L)
            s = pl.ds(off, L)
            rowbase = g * L + lanes
            cx = bx[s] * G1S + G1S
            cy = by[s] * G1S + G1S
            cz = bz[s] * G1S + G1S
            ix = cx.astype(jnp.int32)
            iy = cy.astype(jnp.int32)
            iz = cz.astype(jnp.int32)
            fx = cx - ix.astype(jnp.float32)
            fy = cy - iy.astype(jnp.float32)
            fz = cz - iz.astype(jnp.float32)
            wq = (
                (1.0 - fy) * (1.0 - fx),
                (1.0 - fy) * fx,
                fy * (1.0 - fx),
                fy * fx,
            )
            wz1 = fz
            wz0 = 1.0 - fz
            wA = [wq[q] * wz0 for q in range(4)]
            wB = [wq[q] * wz1 for q in range(4)]
            g1 = []
            for c in range(3):
                terms = []
                for q in range(4):
                    wv = plsc.load_gather(
                        rows, [rowbase, jnp.full((L,), q * 3 + c, jnp.int32)])
                    a, b = plsc.unpack(plsc.bitcast(wv, jnp.bfloat16),
                                       format=plsc.PackFormat.INTERLEAVED)
                    terms.append(wA[q] * a + wB[q] * b)
                g1.append((terms[0] + terms[1]) + (terms[2] + terms[3]))

            cfx = g1[0] * FS + FS
            cfy = g1[1] * FS + FS
            cfz = g1[2] * FS + FS
            jx = cfx.astype(jnp.int32)
            jy = cfy.astype(jnp.int32)
            jz = cfz.astype(jnp.int32)
            ffx = cfx - jx.astype(jnp.float32)
            ffy = cfy - jy.astype(jnp.float32)
            ffz = cfz - jz.astype(jnp.float32)
            xs = (jnp.clip(jx, 0, FM),)
            xs = xs + (jnp.minimum(xs[0] + 1, FM),)
            ys = (jnp.clip(jy, 0, FM),)
            ys = ys + (jnp.minimum(ys[0] + 1, FM),)
            zs = (jnp.clip(jz, 0, FM),)
            zs = zs + (jnp.minimum(zs[0] + 1, FM),)
            wxs = (1.0 - ffx, ffx)
            wys = (1.0 - ffy, ffy)
            wzs = (1.0 - ffz, ffz)
            cells = []
            wfs = []
            for dz in (0, 1):
                for dy in (0, 1):
                    for dx in (0, 1):
                        cells.append(xs[dx] + ys[dy] * NFPD
                                     + zs[dz] * (NFPD * NFPD))
                        wfs.append(wzs[dz] * wys[dy] * wxs[dx])
            for w in range(FEATURE_DIM // 2):
                wcol = jnp.full((L,), w, jnp.int32)
                ta = []
                tb = []
                for k in range(8):
                    wv = plsc.load_gather(ftab, [cells[k], wcol])
                    a, b = plsc.unpack(plsc.bitcast(wv, jnp.bfloat16),
                                       format=plsc.PackFormat.INTERLEAVED)
                    ta.append(wfs[k] * a)
                    tb.append(wfs[k] * b)
                outb[2 * w, s] = _tree8(ta)
                outb[2 * w + 1, s] = _tree8(tb)

    fire_pts(0, 0)
    drain_pts(0, 0)
    idx_compute(0)
    fire_rows(0)
    fire_pts(1, 1)

    def pair_body(p, carry):
        for b in (0, 1):
            u = p * 2 + b
            ob = 1 - b
            drain_pts(u + 1, ob)
            idx_compute(ob)
            fire_rows(ob)
            drain_rows(b)

            @pl.when(p >= 1)
            def _():
                out_desc(u - 2, b, True).wait()

            main_compute(b)
            out_desc(u, b, False)
            fire_pts(u + 2, b)
        return carry
    lax.fori_loop(0, NCHUNK // 2, pair_body, 0)
    drain_rows(0)
    drain_pts(NCHUNK + 1, 1)
    out_desc(NCHUNK - 2, 0, True).wait()
    out_desc(NCHUNK - 1, 1, True).wait()


def _sc_gather(px, py, pz, T1, Fp):
    mesh = plsc.VectorSubcoreMesh(core_axis_name="c", subcore_axis_name="s")
    fn = pl.kernel(
        _sc_body,
        out_type=jax.ShapeDtypeStruct((FEATURE_DIM, N_PTS), jnp.float32),
        mesh=mesh,
        compiler_params=pltpu.CompilerParams(
            needs_layout_passes=False, use_tc_tiling_on_sc=False),
        scratch_types=(
            [pltpu.VMEM((CH,), jnp.float32)] * 6
            + [pltpu.VMEM((CH,), jnp.int32)] * 2
            + [pltpu.VMEM((CH, 16), jnp.int32)] * 2
            + [pltpu.VMEM((FEATURE_DIM, CH), jnp.float32)] * 2
            + [pltpu.VMEM((NFPD ** 3, FEATURE_DIM // 2), jnp.int32)]
            + [pltpu.SemaphoreType.DMA] * 6
        ),
    )
    return fn(px, py, pz, T1, Fp)


def _shift_right(x, sh):
    col = lax.broadcasted_iota(jnp.int32, x.shape, 1)
    return jnp.where(col >= sh, pltpu.roll(x, sh, axis=1), 0.0)


def _cumsum_minor(x):
    for sh in (1, 2, 4, 8, 16, 32, 64):
        x = x + _shift_right(x, sh)
    return x


def _bmm(a, b):
    return lax.dot(a.astype(jnp.bfloat16), b.astype(jnp.bfloat16),
                   preferred_element_type=jnp.float32)


def _mlp_body(fv_ref, d_ref, m_ref, t_ref, Ws1_ref, bs1_ref, Ws2_ref, bs2_ref,
              Wc1a_ref, Wc1b_ref, bc1_ref, Wc2_ref, bc2_ref, out_ref):
    R = RAY_BLOCK
    fv = fv_ref[...].astype(jnp.bfloat16)
    h = jnp.maximum(_bmm(Ws1_ref[...], fv) + bs1_ref[...][:, None], 0.0)
    sigma = jnp.maximum(_bmm(Ws2_ref[...], h) + bs2_ref[...][:, None], 0.0)[0]
    sigma2 = sigma.reshape(R, N_INT)
    m = m_ref[...]
    sigma2 = jnp.where(m, sigma2, 0.0)
    t_ = t_ref[...]
    col = lax.broadcasted_iota(jnp.int32, t_.shape, 1)
    t_next = pltpu.roll(t_, N_INT - 1, axis=1)
    deltas = jnp.where(col < N_INT - 1, t_next - t_, STEP)
    alpha = 1.0 - jnp.exp(-sigma2 * deltas)
    logx = jnp.log(1.0 - alpha + 1e-10)
    trans = jnp.exp(_shift_right(_cumsum_minor(logx), 1))
    abs_light = alpha * trans
    acc = jnp.sum(abs_light, axis=1)
    hc = _bmm(Wc1a_ref[...], fv)
    dpart = Wc1b_ref[...] @ d_ref[...] + bc1_ref[...][:, None]
    hc = hc.reshape(WIDTH, R, N_INT) + dpart[:, :, None]
    hc = jnp.maximum(hc, 0.0).reshape(WIDTH, PB)
    pre = _bmm(Wc2_ref[...], hc) + bc2_ref[...][:, None]
    outs = []
    for cc in range(3):
        rgb_c = jax.nn.sigmoid(pre[cc].reshape(R, N_INT))
        rgb_c = jnp.where(m, rgb_c, 0.0)
        o_c = jnp.sum(abs_light * rgb_c, axis=1) + (1.0 - acc)
        outs.append(o_c[:, None])
    out_ref[...] = jnp.concatenate(outs, axis=1)


def _mlp_composite(FvalsT, dT, mask, t, Ws1, bs1, Ws2, bs2, Wc1, bc1, Wc2,
                   bc2):
    return pl.pallas_call(
        _mlp_body,
        grid=(N_BLOCKS,),
        in_specs=[
            pl.BlockSpec((FEATURE_DIM, PB), lambda i: (0, i)),
            pl.BlockSpec((3, RAY_BLOCK), lambda i: (0, i)),
            pl.BlockSpec((RAY_BLOCK, N_INT), lambda i: (i, 0)),
            pl.BlockSpec((RAY_BLOCK, N_INT), lambda i: (i, 0)),
            pl.BlockSpec((WIDTH, FEATURE_DIM), lambda i: (0, 0)),
            pl.BlockSpec((WIDTH,), lambda i: (0,)),
            pl.BlockSpec((1, WIDTH), lambda i: (0, 0)),
            pl.BlockSpec((1,), lambda i: (0,)),
            pl.BlockSpec((WIDTH, FEATURE_DIM), lambda i: (0, 0)),
            pl.BlockSpec((WIDTH, 3), lambda i: (0, 0)),
            pl.BlockSpec((WIDTH,), lambda i: (0,)),
            pl.BlockSpec((3, WIDTH), lambda i: (0, 0)),
            pl.BlockSpec((3,), lambda i: (0,)),
        ],
        out_specs=pl.BlockSpec((RAY_BLOCK, 3), lambda i: (i, 0)),
        out_shape=jax.ShapeDtypeStruct((BATCH, 3), jnp.float32),
    )(FvalsT, dT, mask, t, Ws1.T, bs1, Ws2.T, bs2, Wc1[:FEATURE_DIM].T,
      Wc1[FEATURE_DIM:].T, bc1, Wc2.T, bc2)


def kernel(rays_o, rays_d, G1, F, Ws1, bs1, Ws2, bs2, Wc1, bc1, Wc2, bc2):
    d = rays_d / jnp.linalg.norm(rays_d, axis=-1, keepdims=True)
    b = jnp.sum(rays_o * d, axis=-1)
    c = jnp.sum(rays_o * rays_o, axis=-1) - RADIUS * RADIUS
    disc = b * b - c
    t_near = jnp.maximum(-b - jnp.sqrt(jnp.maximum(disc, 0.0)), 0.0)
    t = t_near[:, None] + jnp.arange(N_INT, dtype=jnp.float32)[None, :] * STEP
    pts = rays_o[:, None, :] + t[..., None] * d[:, None, :]
    mask = (disc[:, None] > 0.0) & (jnp.linalg.norm(pts, axis=-1) <= RADIUS)
    pts = pts / RADIUS
    px = pts[..., 0].reshape(-1)
    py = pts[..., 1].reshape(-1)
    pz = pts[..., 2].reshape(-1)

    T1 = _build_g1_table(G1)
    Fp = _build_f_table(F)
    FvalsT = _sc_gather(px, py, pz, T1, Fp)

    return _mlp_composite(FvalsT, d.T, mask, t,
                          Ws1, bs1, Ws2, bs2, Wc1, bc1, Wc2, bc2)


# speedup vs baseline: 1.9525x; 1.0782x over previous
import jax
import jax.numpy as jnp
from jax import lax
from jax.experimental import pallas as pl
from jax.experimental.pallas import tpu as pltpu
from jax.experimental.pallas import tpu_sc as plsc

RESOLUTION = 128
FEATURE_DIM = 32
NFPD = 16
RADIUS = 1.0
N_INT = 128
STEP = 0.01
WIDTH = 64
BATCH = 4096
N_PTS = BATCH * N_INT

NC = 2
NS = 16
L = 16
NW = NC * NS
PW = N_PTS // NW
CH = 512
NCHUNK = PW // CH
GRP = CH // L

RAY_BLOCK = 128
N_BLOCKS = BATCH // RAY_BLOCK
PB = RAY_BLOCK * N_INT


def _tree8(vs):
    return (vs[0] + vs[1]) + (vs[2] + vs[3]) + ((vs[4] + vs[5]) + (vs[6] + vs[7]))


RP_Y = 16
RP_UNITS = RESOLUTION * (RESOLUTION // RP_Y)
RP_PER_W = RP_UNITS // NW
RP_ROWS = RP_Y * RESOLUTION


def _repack_body(g1_h, t1_h, slabA, slabB, outA, outB, ss0, ss1, so0, so1):
    SLAB = (slabA, slabB)
    OUTB = (outA, outB)
    SS = (ss0, ss1)
    SO = (so0, so1)
    wid = lax.axis_index("s") * NC + lax.axis_index("c")
    lanes = lax.iota(jnp.int32, L)
    M = RESOLUTION - 1

    def params(u):
        ug = wid * RP_PER_W + jnp.minimum(u, RP_PER_W - 1)
        z = ug // (RESOLUTION // RP_Y)
        y0 = (ug % (RESOLUTION // RP_Y)) * RP_Y
        return z, y0, jnp.minimum(y0, RESOLUTION - (RP_Y + 1))

    def slab_descs(u, b, make):
        z, y0, yb = params(u)
        z1 = jnp.minimum(z + 1, M)
        f = pltpu.make_async_copy if make else pltpu.async_copy
        return [f(g1_h.at[c, zc, pl.ds(yb, RP_Y + 1)],
                  SLAB[b].at[c, dz], SS[b])
                for c in range(3) for dz, zc in ((0, z), (1, z1))]

    def drain_slabs(u, b):
        for dsc in slab_descs(u, b, True):
            dsc.wait()

    def out_desc(u, b, make):
        z, y0, _ = params(u)
        f = pltpu.make_async_copy if make else pltpu.async_copy
        return f(OUTB[b], t1_h.at[pl.ds(z * (RESOLUTION * RESOLUTION)
                                        + y0 * RESOLUTION, RP_ROWS)], SO[b])

    def compute(u, b):
        slab, outb = SLAB[b], OUTB[b]
        z, y0, yb = params(u)

        @plsc.parallel_loop(0, RP_Y, unroll=2)
        def row_body(j):
            yj = y0 + j
            i0 = jnp.minimum(yj, M) - yb
            i1 = jnp.minimum(yj + 1, M) - yb
            for xg in range(RESOLUTION // L):
                x_lin = pl.ds(xg * L, L)
                xi1 = jnp.minimum(xg * L + lanes + 1, M)
                orow = j * RESOLUTION + xg * L + lanes
                for dy, iy in ((0, i0), (1, i1)):
                    iyv = jnp.full((L,), iy, jnp.int32)
                    for dx in (0, 1):
                        q = dy * 2 + dx
                        for c in range(3):
                            if dx == 0:
                                a = slab[c, 0, iy, x_lin]
                                bb = slab[c, 1, iy, x_lin]
                            else:
                                cv = jnp.full((L,), c, jnp.int32)
                                z0v = jnp.full((L,), 0, jnp.int32)
                                z1v = jnp.full((L,), 1, jnp.int32)
                                a = plsc.load_gather(slab, [cv, z0v, iyv, xi1])
                                bb = plsc.load_gather(slab, [cv, z1v, iyv, xi1])
                            pk = plsc.bitcast(
                                plsc.pack(a, bb,
                                          format=plsc.PackFormat.INTERLEAVED),
                                jnp.int32)
                            plsc.store_scatter(
                                outb,
                                [orow, jnp.full((L,), q * 3 + c, jnp.int32)],
                                pk)

    slab_descs(0, 0, False)
    slab_descs(1, 1, False)

    def pair_body(p, carry):
        for b in (0, 1):
            u = p * 2 + b
            drain_slabs(u, b)

            @pl.when(p >= 1)
            def _():
                out_desc(u - 2, b, True).wait()

            compute(u, b)
            out_desc(u, b, False)
            slab_descs(u + 2, b, False)
        return carry
    lax.fori_loop(0, RP_PER_W // 2, pair_body, 0)
    drain_slabs(RP_PER_W, 0)
    drain_slabs(RP_PER_W + 1, 1)
    out_desc(RP_PER_W - 2, 0, True).wait()
    out_desc(RP_PER_W - 1, 1, True).wait()


def _build_g1_table(G1):
    mesh = plsc.VectorSubcoreMesh(core_axis_name="c", subcore_axis_name="s")
    fn = pl.kernel(
        _repack_body,
        out_type=jax.ShapeDtypeStruct((RESOLUTION ** 3, 16), jnp.int32),
        mesh=mesh,
        compiler_params=pltpu.CompilerParams(
            needs_layout_passes=False, use_tc_tiling_on_sc=False),
        scratch_types=(
            [pltpu.VMEM((3, 2, RP_Y + 1, RESOLUTION), jnp.float32)] * 2
            + [pltpu.VMEM((RP_ROWS, 16), jnp.int32)] * 2
            + [pltpu.SemaphoreType.DMA] * 4
        ),
    )
    return fn(G1)


def _build_f_table(F):
    f = jnp.transpose(F, (1, 2, 3, 0)).reshape(NFPD ** 3, FEATURE_DIM)
    fb = f.astype(jnp.bfloat16).reshape(NFPD ** 3, FEATURE_DIM // 2, 2)
    return lax.bitcast_convert_type(fb, jnp.int32)


def _sc_body(px_h, py_h, pz_h, t1_h, fp_h, out_h,
             bx0, by0, bz0, bx1, by1, bz1, idx0, idx1, rows0, rows1,
             outb0, outb1, ftab, sp0, sp1, sr0, sr1, so0, so1,
             *, pw, nchunk):
    BX = (bx0, bx1)
    BY = (by0, by1)
    BZ = (bz0, bz1)
    IDX = (idx0, idx1)
    ROWS = (rows0, rows1)
    OUTB = (outb0, outb1)
    SP = (sp0, sp1)
    SR = (sr0, sr1)
    SO = (so0, so1)
    wid = lax.axis_index("s") * NC + lax.axis_index("c")
    base_w = wid * pw
    pltpu.sync_copy(fp_h, ftab)
    lanes = lax.iota(jnp.int32, L)
    G1M = RESOLUTION - 1
    G1S = (RESOLUTION - 1) * 0.5
    FM = NFPD - 1
    FS = (NFPD - 1) * 0.5

    def cbase(u):
        return base_w + jnp.minimum(u, nchunk - 1) * CH

    def pts_descs(u, b, make):
        base = cbase(u)
        f = pltpu.make_async_copy if make else pltpu.async_copy
        return [f(px_h.at[pl.ds(base, CH)], BX[b], SP[b]),
                f(py_h.at[pl.ds(base, CH)], BY[b], SP[b]),
                f(pz_h.at[pl.ds(base, CH)], BZ[b], SP[b])]

    def fire_pts(u, b):
        pts_descs(u, b, False)

    def drain_pts(u, b):
        for dsc in pts_descs(u, b, True):
            dsc.wait()

    def idx_compute(b):
        bx, by, bz, idxb = BX[b], BY[b], BZ[b], IDX[b]

        @plsc.parallel_loop(0, GRP, unroll=4)
        def idx_body(g):
            s = pl.ds(pl.multiple_of(g * L, L), L)
            x0 = jnp.clip((bx[s] * G1S + G1S).astype(jnp.int32), 0, G1M)
            y0 = jnp.clip((by[s] * G1S + G1S).astype(jnp.int32), 0, G1M)
            z0 = jnp.clip((bz[s] * G1S + G1S).astype(jnp.int32), 0, G1M)
            idxb[s] = x0 + y0 * RESOLUTION + z0 * (RESOLUTION * RESOLUTION)

    def row_descs(b, make):
        f = pltpu.make_async_copy if make else pltpu.async_copy
        return [f(t1_h.at[IDX[b].at[pl.ds(j * 128, 128)]],
                  ROWS[b].at[pl.ds(j * 128, 128)], SR[b])
                for j in range(CH // 128)]

    def fire_rows(b):
        row_descs(b, False)

    def drain_rows(b):
        for j in range(CH // 128):
            pltpu.make_async_copy(t1_h.at[pl.ds(0, 128)],
                                  ROWS[b].at[pl.ds(j * 128, 128)],
                                  SR[b]).wait()

    def out_desc(u, b, make):
        f = pltpu.make_async_copy if make else pltpu.async_copy
        return f(OUTB[b], out_h.at[:, pl.ds(cbase(u), CH)], SO[b])

    def main_compute(b):
        bx, by, bz, rows, outb = BX[b], BY[b], BZ[b], ROWS[b], OUTB[b]

        @plsc.parallel_loop(0, GRP, unroll=2)
        def grp_body(g):
            off = pl.multiple_of(g * L, L)
            s = pl.ds(off, L)
            rowbase = g * L + lanes
            cx = bx[s] * G1S + G1S
            cy = by[s] * G1S + G1S
            cz = bz[s] * G1S + G1S
            ix = cx.astype(jnp.int32)
            iy = cy.astype(jnp.int32)
            iz = cz.astype(jnp.int32)
            fx = cx - ix.astype(jnp.float32)
            fy = cy - iy.astype(jnp.float32)
            fz = cz - iz.astype(jnp.float32)
            wq = (
                (1.0 - fy) * (1.0 - fx),
                (1.0 - fy) * fx,
                fy * (1.0 - fx),
                fy * fx,
            )
            wz1 = fz
            wz0 = 1.0 - fz
            wA = [wq[q] * wz0 for q in range(4)]
            wB = [wq[q] * wz1 for q in range(4)]
            g1 = []
            for c in range(3):
                terms = []
                for q in range(4):
                    wv = plsc.load_gather(
                        rows, [rowbase, jnp.full((L,), q * 3 + c, jnp.int32)])
                    a, b = plsc.unpack(plsc.bitcast(wv, jnp.bfloat16),
                                       format=plsc.PackFormat.INTERLEAVED)
                    terms.append(wA[q] * a + wB[q] * b)
                g1.append((terms[0] + terms[1]) + (terms[2] + terms[3]))

            cfx = g1[0] * FS + FS
            cfy = g1[1] * FS + FS
            cfz = g1[2] * FS + FS
            jx = cfx.astype(jnp.int32)
            jy = cfy.astype(jnp.int32)
            jz = cfz.astype(jnp.int32)
            ffx = cfx - jx.astype(jnp.float32)
            ffy = cfy - jy.astype(jnp.float32)
            ffz = cfz - jz.astype(jnp.float32)
            xs = (jnp.clip(jx, 0, FM),)
            xs = xs + (jnp.minimum(xs[0] + 1, FM),)
            ys = (jnp.clip(jy, 0, FM),)
            ys = ys + (jnp.minimum(ys[0] + 1, FM),)
            zs = (jnp.clip(jz, 0, FM),)
            zs = zs + (jnp.minimum(zs[0] + 1, FM),)
            wxs = (1.0 - ffx, ffx)
            wys = (1.0 - ffy, ffy)
            wzs = (1.0 - ffz, ffz)
            cells = []
            wfs = []
            for dz in (0, 1):
                for dy in (0, 1):
                    for dx in (0, 1):
                        cells.append(xs[dx] + ys[dy] * NFPD
                                     + zs[dz] * (NFPD * NFPD))
                        wfs.append(wzs[dz] * wys[dy] * wxs[dx])
            for w in range(FEATURE_DIM // 2):
                wcol = jnp.full((L,), w, jnp.int32)
                ta = []
                tb = []
                for k in range(8):
                    wv = plsc.load_gather(ftab, [cells[k], wcol])
                    a, b = plsc.unpack(plsc.bitcast(wv, jnp.bfloat16),
                                       format=plsc.PackFormat.INTERLEAVED)
                    ta.append(wfs[k] * a)
                    tb.append(wfs[k] * b)
                outb[2 * w, s] = _tree8(ta)
                outb[2 * w + 1, s] = _tree8(tb)

    fire_pts(0, 0)
    drain_pts(0, 0)
    idx_compute(0)
    fire_rows(0)
    fire_pts(1, 1)

    def pair_body(p, carry):
        for b in (0, 1):
            u = p * 2 + b
            ob = 1 - b
            drain_pts(u + 1, ob)
            idx_compute(ob)
            fire_rows(ob)
            drain_rows(b)

            @pl.when(p >= 1)
            def _():
                out_desc(u - 2, b, True).wait()

            main_compute(b)
            out_desc(u, b, False)
            fire_pts(u + 2, b)
        return carry
    lax.fori_loop(0, nchunk // 2, pair_body, 0)
    drain_rows(0)
    drain_pts(nchunk + 1, 1)
    out_desc(nchunk - 2, 0, True).wait()
    out_desc(nchunk - 1, 1, True).wait()


def _sc_gather(px, py, pz, T1, Fp):
    import functools
    npts = px.shape[0]
    mesh = plsc.VectorSubcoreMesh(core_axis_name="c", subcore_axis_name="s")
    fn = pl.kernel(
        functools.partial(_sc_body, pw=npts // NW, nchunk=npts // NW // CH),
        out_type=jax.ShapeDtypeStruct((FEATURE_DIM, npts), jnp.float32),
        mesh=mesh,
        compiler_params=pltpu.CompilerParams(
            needs_layout_passes=False, use_tc_tiling_on_sc=False),
        scratch_types=(
            [pltpu.VMEM((CH,), jnp.float32)] * 6
            + [pltpu.VMEM((CH,), jnp.int32)] * 2
            + [pltpu.VMEM((CH, 16), jnp.int32)] * 2
            + [pltpu.VMEM((FEATURE_DIM, CH), jnp.float32)] * 2
            + [pltpu.VMEM((NFPD ** 3, FEATURE_DIM // 2), jnp.int32)]
            + [pltpu.SemaphoreType.DMA] * 6
        ),
    )
    return fn(px, py, pz, T1, Fp)


def _shift_right(x, sh):
    col = lax.broadcasted_iota(jnp.int32, x.shape, 1)
    return jnp.where(col >= sh, pltpu.roll(x, sh, axis=1), 0.0)


def _cumsum_minor(x):
    for sh in (1, 2, 4, 8, 16, 32, 64):
        x = x + _shift_right(x, sh)
    return x


def _bmm(a, b):
    return lax.dot(a.astype(jnp.bfloat16), b.astype(jnp.bfloat16),
                   preferred_element_type=jnp.float32)


def _mlp_body(fv_ref, d_ref, m_ref, t_ref, Ws1_ref, bs1_ref, Ws2_ref, bs2_ref,
              Wc1a_ref, Wc1b_ref, bc1_ref, Wc2_ref, bc2_ref, out_ref):
    R = RAY_BLOCK
    fv = fv_ref[...].astype(jnp.bfloat16)
    h = jnp.maximum(_bmm(Ws1_ref[...], fv) + bs1_ref[...][:, None], 0.0)
    sigma = jnp.maximum(_bmm(Ws2_ref[...], h) + bs2_ref[...][:, None], 0.0)[0]
    sigma2 = sigma.reshape(R, N_INT)
    m = m_ref[...]
    sigma2 = jnp.where(m, sigma2, 0.0)
    t_ = t_ref[...]
    col = lax.broadcasted_iota(jnp.int32, t_.shape, 1)
    t_next = pltpu.roll(t_, N_INT - 1, axis=1)
    deltas = jnp.where(col < N_INT - 1, t_next - t_, STEP)
    alpha = 1.0 - jnp.exp(-sigma2 * deltas)
    logx = jnp.log(1.0 - alpha + 1e-10)
    trans = jnp.exp(_shift_right(_cumsum_minor(logx), 1))
    abs_light = alpha * trans
    acc = jnp.sum(abs_light, axis=1)
    hc = _bmm(Wc1a_ref[...], fv)
    dpart = Wc1b_ref[...] @ d_ref[...] + bc1_ref[...][:, None]
    hc = hc.reshape(WIDTH, R, N_INT) + dpart[:, :, None]
    hc = jnp.maximum(hc, 0.0).reshape(WIDTH, PB)
    pre = _bmm(Wc2_ref[...], hc) + bc2_ref[...][:, None]
    outs = []
    for cc in range(3):
        rgb_c = jax.nn.sigmoid(pre[cc].reshape(R, N_INT))
        rgb_c = jnp.where(m, rgb_c, 0.0)
        o_c = jnp.sum(abs_light * rgb_c, axis=1) + (1.0 - acc)
        outs.append(o_c[:, None])
    out_ref[...] = jnp.concatenate(outs, axis=1)


def _mlp_composite(FvalsT, dT, mask, t, Ws1, bs1, Ws2, bs2, Wc1, bc1, Wc2,
                   bc2):
    nrays = mask.shape[0]
    return pl.pallas_call(
        _mlp_body,
        grid=(nrays // RAY_BLOCK,),
        in_specs=[
            pl.BlockSpec((FEATURE_DIM, PB), lambda i: (0, i)),
            pl.BlockSpec((3, RAY_BLOCK), lambda i: (0, i)),
            pl.BlockSpec((RAY_BLOCK, N_INT), lambda i: (i, 0)),
            pl.BlockSpec((RAY_BLOCK, N_INT), lambda i: (i, 0)),
            pl.BlockSpec((WIDTH, FEATURE_DIM), lambda i: (0, 0)),
            pl.BlockSpec((WIDTH,), lambda i: (0,)),
            pl.BlockSpec((1, WIDTH), lambda i: (0, 0)),
            pl.BlockSpec((1,), lambda i: (0,)),
            pl.BlockSpec((WIDTH, FEATURE_DIM), lambda i: (0, 0)),
            pl.BlockSpec((WIDTH, 3), lambda i: (0, 0)),
            pl.BlockSpec((WIDTH,), lambda i: (0,)),
            pl.BlockSpec((3, WIDTH), lambda i: (0, 0)),
            pl.BlockSpec((3,), lambda i: (0,)),
        ],
        out_specs=pl.BlockSpec((RAY_BLOCK, 3), lambda i: (i, 0)),
        out_shape=jax.ShapeDtypeStruct((nrays, 3), jnp.float32),
    )(FvalsT, dT, mask, t, Ws1.T, bs1, Ws2.T, bs2, Wc1[:FEATURE_DIM].T,
      Wc1[FEATURE_DIM:].T, bc1, Wc2.T, bc2)


def kernel(rays_o, rays_d, G1, F, Ws1, bs1, Ws2, bs2, Wc1, bc1, Wc2, bc2):
    d = rays_d / jnp.linalg.norm(rays_d, axis=-1, keepdims=True)
    b = jnp.sum(rays_o * d, axis=-1)
    c = jnp.sum(rays_o * rays_o, axis=-1) - RADIUS * RADIUS
    disc = b * b - c
    t_near = jnp.maximum(-b - jnp.sqrt(jnp.maximum(disc, 0.0)), 0.0)
    t = t_near[:, None] + jnp.arange(N_INT, dtype=jnp.float32)[None, :] * STEP
    pts = rays_o[:, None, :] + t[..., None] * d[:, None, :]
    mask = (disc[:, None] > 0.0) & (jnp.linalg.norm(pts, axis=-1) <= RADIUS)
    pts = pts / RADIUS
    px = pts[..., 0].reshape(-1)
    py = pts[..., 1].reshape(-1)
    pz = pts[..., 2].reshape(-1)

    T1 = _build_g1_table(G1)
    Fp = _build_f_table(F)

    half = N_PTS // 2
    hr = BATCH // 2
    dT = d.T
    outs = []
    for i in (0, 1):
        s = slice(i * half, (i + 1) * half)
        r = slice(i * hr, (i + 1) * hr)
        FvT = _sc_gather(px[s], py[s], pz[s], T1, Fp)
        outs.append(_mlp_composite(FvT, dT[:, r], mask[r], t[r],
                                   Ws1, bs1, Ws2, bs2, Wc1, bc1, Wc2, bc2))
    return jnp.concatenate(outs, axis=0)
